# baseline (speedup 1.0000x reference)
import numpy as np
import jax
import jax.numpy as jnp
from jax import lax
from jax.experimental import pallas as pl
from jax.experimental.pallas import tpu as pltpu
from jax.experimental.pallas import tpu_sc as plsc

D = 128
P = D * (D + 1) // 2
NK = D // 2 + 1
G = 512
N = 10000
NPAD = 10240
BT = 640

NTILES = 16
ROWS_PT = NPAD // NTILES
IDXROWS_PT = ROWS_PT // 128
GH = G // 2
GACC = 384
ZROWS_PT = GACC // NTILES

_SELU_SCALE = 1.0507009873554804934193349852946
_SELU_ALPHA = 1.6732632423543772848170429916717
_RSQRT_P = 1.0 / np.sqrt(np.float32(P))
_QROOT2 = np.float32(2.0) ** 0.25

_iu, _ju = np.triu_indices(D)
_pair = np.zeros((D, D), np.int32)
_pair[_iu, _ju] = np.arange(P, dtype=np.int32)
_pair[_ju, _iu] = np.arange(P, dtype=np.int32)
_ii = np.tile(np.arange(D)[None, :], (NK, 1))
_jj = (_ii + np.arange(NK)[:, None]) % D
_PIDX = _pair[_ii, _jj]
_MASKK = np.ones((NK, D), np.float32)
_MASKK[NK - 1, D // 2:] = 0.0
_BIGIDX = np.concatenate([_PIDX, _pair], axis=0)

_TILE_OF = (np.arange(NPAD) // ROWS_PT).astype(np.int32)


def _dense_body(f_ref, idx_ref, wk_ref, a2_ref, y_ref, z_ref, zacc_ref):
    i = pl.program_id(0)
    nsteps = pl.num_programs(0)
    F = f_ref[...]

    Fa = jnp.abs(F)
    qf1 = jnp.sum(jnp.dot(F, a2_ref[...], preferred_element_type=jnp.float32)
                  * F, axis=1, keepdims=True)
    qf2 = jnp.sum(jnp.dot(Fa, a2_ref[...], preferred_element_type=jnp.float32)
                  * Fa, axis=1, keepdims=True)

    parts = []
    for cch in range(BT // 64):
        Fsc = f_ref[pl.ds(64 * cch, 64), :]
        acc = jnp.zeros((64, D), jnp.float32)
        for k in range(1, NK):
            Frc = jnp.concatenate([Fsc[:, k:], Fsc[:, :k]], axis=1)
            T = Fsc * Frc
            E = jnp.exp2(jnp.minimum(T, 0.0))
            acc = acc + E * wk_ref[k:k + 1, :]
        parts.append(jnp.sum(acc, axis=1, keepdims=True))
    accsum = jnp.concatenate(parts, axis=0)
    const = jnp.sum(wk_ref[...])
    logit = qf1 + qf2 + accsum - const
    row = i * BT + lax.broadcasted_iota(jnp.int32, (BT, 1), 0)
    e = jnp.where(row < N, jnp.exp(logit), 0.0)
    y_ref[...] = F * (e * np.float32(1.0 / (_QROOT2 * np.sqrt(np.log2(np.e)))))

    idx = idx_ref[...].reshape(1, BT)
    onehot_t = (lax.broadcasted_iota(jnp.int32, (G, BT), 0) == idx
                ).astype(jnp.float32)

    @pl.when(i == 0)
    def _init():
        zacc_ref[...] = jnp.zeros_like(zacc_ref)

    zacc_ref[...] += jnp.dot(onehot_t, jnp.broadcast_to(e, (BT, D)),
                             preferred_element_type=jnp.float32)

    @pl.when(i == nsteps - 1)
    def _finish():
        z_ref[...] = zacc_ref[...]


def _dense_stage(node_ft_pad, idx3, wk, a2):
    return pl.pallas_call(
        _dense_body,
        grid=(NPAD // BT,),
        in_specs=[
            pl.BlockSpec((BT, D), lambda i: (i, 0)),
            pl.BlockSpec((1, 1, BT), lambda i: (i, 0, 0)),
            pl.BlockSpec((72, D), lambda i: (0, 0)),
            pl.BlockSpec((D, D), lambda i: (0, 0)),
        ],
        out_specs=[
            pl.BlockSpec((BT, D), lambda i: (i, 0)),
            pl.BlockSpec((G, D), lambda i: (0, 0)),
        ],
        out_shape=[
            jax.ShapeDtypeStruct((NPAD, D), jnp.float32),
            jax.ShapeDtypeStruct((G, D), jnp.float32),
        ],
        scratch_shapes=[pltpu.VMEM((G, D), jnp.float32)],
    )(node_ft_pad, idx3, wk, a2)


def _sc_body(y_hbm, idx_hbm, z_hbm, zy_hbm, out_hbm,
             y_v, idx_v, accy_gather, z_gather, out_v, accy):
    c = lax.axis_index("c")
    s = lax.axis_index("s")
    glo = c * GH

    pltpu.sync_copy(zy_hbm.at[pl.ds(s * ZROWS_PT, ZROWS_PT)],
                    accy.at[pl.ds(s * ZROWS_PT, ZROWS_PT)])
    pltpu.sync_copy(idx_hbm.at[c * NTILES + s], idx_v)

    plsc.subcore_barrier()

    for j in range(IDXROWS_PT):
        pltpu.sync_copy(y_hbm.at[pl.ds(s * ROWS_PT + 128 * j, 128)], y_v)
        pltpu.sync_copy(y_v, accy.at[idx_v.at[j]], add=True)

    plsc.subcore_barrier()

    pltpu.sync_copy(accy.at[pl.ds(s * 16, 16)], accy_gather)
    pltpu.sync_copy(z_hbm.at[pl.ds(glo + s * 16, 16)], z_gather)
    for r in range(16):
        for q in range(8):
            z = jnp.maximum(z_gather[r, pl.ds(16 * q, 16)], 1e-30)
            out_v[r, pl.ds(16 * q, 16)] = accy_gather[r, pl.ds(16 * q, 16)] / z
    pltpu.sync_copy(out_v, out_hbm.at[pl.ds(glo + s * 16, 16)])


def _sc_stage():
    return pl.kernel(
        _sc_body,
        mesh=plsc.VectorSubcoreMesh(core_axis_name="c", subcore_axis_name="s"),
        out_type=jax.ShapeDtypeStruct((G, D), jnp.float32),
        scratch_types=[
            pltpu.VMEM((128, D), jnp.float32),
            pltpu.VMEM((IDXROWS_PT, 128), jnp.int32),
            pltpu.VMEM((16, D), jnp.float32),
            pltpu.VMEM((16, D), jnp.float32),
            pltpu.VMEM((16, D), jnp.float32),
            pltpu.VMEM_SHARED((GACC, D), jnp.float32),
        ],
    )


def _prep_weights(W):
    big = W[jnp.asarray(_BIGIDX)]
    mask0 = np.copy(_MASKK)
    mask0[0] = 0.0
    wk = big[:NK] * jnp.asarray(mask0) * (_SELU_SCALE * _SELU_ALPHA
                                          * _RSQRT_P)
    wk = jnp.pad(wk, ((0, 72 - NK), (0, 0)))
    coefm = jnp.asarray(np.where(np.eye(D, dtype=bool), 2.0,
                                 np.sqrt(2.0)).astype(np.float32))
    a2 = big[NK:] * coefm * np.float32(_SELU_SCALE * _RSQRT_P / 4.0
                                       / (np.sqrt(2.0) * np.log2(np.e)))
    return wk, a2


def kernel(node_ft, batch_index, num_graphs, W):
    wk, a2 = _prep_weights(W)
    f_pad = jnp.pad(node_ft * np.float32(_QROOT2 * np.sqrt(np.log2(np.e))),
                    ((0, NPAD - N), (0, 0)))
    idx = batch_index.astype(jnp.int32)
    idx_pad = jnp.pad(idx, (0, NPAD - N), constant_values=G)
    idx3 = jnp.where(idx_pad < G, idx_pad, 0).reshape(NPAD // BT, 1, BT)

    y, z = _dense_stage(f_pad, idx3, wk, a2)

    tile_of = jnp.asarray(_TILE_OF)
    parts = []
    for c in range(2):
        loc = idx_pad - c * GH
        ok = (loc >= 0) & (loc < GH)
        parts.append(jnp.where(ok, loc, GH + tile_of))
    idx6 = jnp.stack(parts).reshape(2 * NTILES, IDXROWS_PT, 128)

    zy = jnp.zeros((GACC, D), jnp.float32)
    out = _sc_stage()(y, idx6, z, zy)

    valid = jnp.arange(G) < num_graphs
    return jnp.where(valid[:, None], out, jnp.zeros_like(out))


# speedup vs baseline: 1.6699x; 1.6699x over previous
import numpy as np
import jax
import jax.numpy as jnp
from jax import lax
from jax.experimental import pallas as pl
from jax.experimental.pallas import tpu as pltpu
from jax.experimental.pallas import tpu_sc as plsc

D = 128
P = D * (D + 1) // 2
NK = D // 2 + 1
G = 512
N = 10000
NPAD = 10240
BT = 640

NTILES = 16
ROWS_PT = NPAD // NTILES
IDXROWS_PT = ROWS_PT // 128
GH = G // 2
GACC = 384
ZROWS_PT = GACC // NTILES

_SELU_SCALE = 1.0507009873554804934193349852946
_SELU_ALPHA = 1.6732632423543772848170429916717
_RSQRT_P = 1.0 / np.sqrt(np.float32(P))
_QROOT2 = np.float32(2.0) ** 0.25

_iu, _ju = np.triu_indices(D)
_pair = np.zeros((D, D), np.int32)
_pair[_iu, _ju] = np.arange(P, dtype=np.int32)
_pair[_ju, _iu] = np.arange(P, dtype=np.int32)
_ii = np.tile(np.arange(D)[None, :], (NK, 1))
_jj = (_ii + np.arange(NK)[:, None]) % D
_PIDX = _pair[_ii, _jj]
_MASKK = np.ones((NK, D), np.float32)
_MASKK[NK - 1, D // 2:] = 0.0
_BIGIDX = np.concatenate([_PIDX, _pair], axis=0)

_TILE_OF = (np.arange(NPAD) // ROWS_PT).astype(np.int32)


def _dense_body(f_ref, idx_ref, wk_ref, a2_ref, y_ref, z_ref, zacc_ref):
    i = pl.program_id(0)
    nsteps = pl.num_programs(0)
    F = f_ref[...]

    Fa = jnp.abs(F)
    qf1 = jnp.sum(jnp.dot(F, a2_ref[...], preferred_element_type=jnp.float32)
                  * F, axis=1, keepdims=True)
    qf2 = jnp.sum(jnp.dot(Fa, a2_ref[...], preferred_element_type=jnp.float32)
                  * Fa, axis=1, keepdims=True)

    parts = []
    for cch in range(BT // 64):
        Fsc = f_ref[pl.ds(64 * cch, 64), :]
        acc = jnp.zeros((64, D), jnp.float32)
        for k in range(1, NK):
            Frc = jnp.concatenate([Fsc[:, k:], Fsc[:, :k]], axis=1)
            T = Fsc * Frc
            E = jnp.exp2(jnp.minimum(T, 0.0))
            acc = acc + E * wk_ref[k:k + 1, :]
        parts.append(jnp.sum(acc, axis=1, keepdims=True))
    accsum = jnp.concatenate(parts, axis=0)
    const = jnp.sum(wk_ref[...])
    logit = qf1 + qf2 + accsum - const
    row = i * BT + lax.broadcasted_iota(jnp.int32, (BT, 1), 0)
    e = jnp.where(row < N, jnp.exp(logit), 0.0)
    y_ref[...] = F * (e * np.float32(1.0 / (_QROOT2 * np.sqrt(np.log2(np.e)))))

    idx = idx_ref[...].reshape(1, BT)
    onehot_t = (lax.broadcasted_iota(jnp.int32, (G, BT), 0) == idx
                ).astype(jnp.float32)

    @pl.when(i == 0)
    def _init():
        zacc_ref[...] = jnp.zeros_like(zacc_ref)

    zacc_ref[...] += jnp.dot(onehot_t, jnp.broadcast_to(e, (BT, D)),
                             preferred_element_type=jnp.float32)

    @pl.when(i == nsteps - 1)
    def _finish():
        z_ref[...] = zacc_ref[...]


def _dense_stage(node_ft_pad, idx3, wk, a2):
    return pl.pallas_call(
        _dense_body,
        grid=(NPAD // BT,),
        in_specs=[
            pl.BlockSpec((BT, D), lambda i: (i, 0)),
            pl.BlockSpec((1, 1, BT), lambda i: (i, 0, 0)),
            pl.BlockSpec((72, D), lambda i: (0, 0)),
            pl.BlockSpec((D, D), lambda i: (0, 0)),
        ],
        out_specs=[
            pl.BlockSpec((BT, D), lambda i: (i, 0)),
            pl.BlockSpec((G, D), lambda i: (0, 0)),
        ],
        out_shape=[
            jax.ShapeDtypeStruct((NPAD, D), jnp.float32),
            jax.ShapeDtypeStruct((G, D), jnp.float32),
        ],
        scratch_shapes=[pltpu.VMEM((G, D), jnp.float32)],
    )(node_ft_pad, idx3, wk, a2)


def _sc_body(y_hbm, idx_hbm, z_hbm, zy_hbm, out_hbm,
             y_v, idx_v, accy_gather, z_gather, out_v, accy):
    c = lax.axis_index("c")
    s = lax.axis_index("s")
    glo = c * GH

    pltpu.sync_copy(zy_hbm.at[pl.ds(s * ZROWS_PT, ZROWS_PT)],
                    accy.at[pl.ds(s * ZROWS_PT, ZROWS_PT)])
    pltpu.sync_copy(idx_hbm.at[c * NTILES + s], idx_v)

    plsc.subcore_barrier()

    for j in range(IDXROWS_PT):
        pltpu.sync_copy(y_hbm.at[pl.ds(s * ROWS_PT + 128 * j, 128)], y_v)
        pltpu.sync_copy(y_v, accy.at[idx_v.at[j]], add=True)

    plsc.subcore_barrier()

    pltpu.sync_copy(accy.at[pl.ds(s * 16, 16)], accy_gather)
    pltpu.sync_copy(z_hbm.at[pl.ds(glo + s * 16, 16)], z_gather)
    for r in range(16):
        for q in range(8):
            z = jnp.maximum(z_gather[r, pl.ds(16 * q, 16)], 1e-30)
            out_v[r, pl.ds(16 * q, 16)] = accy_gather[r, pl.ds(16 * q, 16)] / z
    pltpu.sync_copy(out_v, out_hbm.at[pl.ds(glo + s * 16, 16)])


def _sc_stage():
    return pl.kernel(
        _sc_body,
        mesh=plsc.VectorSubcoreMesh(core_axis_name="c", subcore_axis_name="s"),
        out_type=jax.ShapeDtypeStruct((G, D), jnp.float32),
        scratch_types=[
            pltpu.VMEM((128, D), jnp.float32),
            pltpu.VMEM((IDXROWS_PT, 128), jnp.int32),
            pltpu.VMEM((16, D), jnp.float32),
            pltpu.VMEM((16, D), jnp.float32),
            pltpu.VMEM((16, D), jnp.float32),
            pltpu.VMEM_SHARED((GACC, D), jnp.float32),
        ],
    )


def _prep_weights(W):
    mask0 = np.copy(_MASKK)
    mask0[0] = 0.0
    wk = W[_PIDX] * jnp.asarray(mask0) * (_SELU_SCALE * _SELU_ALPHA
                                          * _RSQRT_P)
    wk = jnp.pad(wk, ((0, 72 - NK), (0, 0)))
    coefm = jnp.asarray(np.where(np.eye(D, dtype=bool), 2.0,
                                 np.sqrt(2.0)).astype(np.float32))
    a2 = W[jnp.asarray(_pair)] * coefm * np.float32(
        _SELU_SCALE * _RSQRT_P / 4.0 / (np.sqrt(2.0) * np.log2(np.e)))
    return wk, a2


def kernel(node_ft, batch_index, num_graphs, W):
    wk, a2 = _prep_weights(W)
    f_pad = jnp.pad(node_ft * np.float32(_QROOT2 * np.sqrt(np.log2(np.e))),
                    ((0, NPAD - N), (0, 0)))
    idx = batch_index.astype(jnp.int32)
    idx_pad = jnp.pad(idx, (0, NPAD - N), constant_values=G)
    idx3 = jnp.where(idx_pad < G, idx_pad, 0).reshape(NPAD // BT, 1, BT)

    y, z = _dense_stage(f_pad, idx3, wk, a2)

    tile_of = jnp.asarray(_TILE_OF)
    parts = []
    for c in range(2):
        loc = idx_pad - c * GH
        ok = (loc >= 0) & (loc < GH)
        parts.append(jnp.where(ok, loc, GH + tile_of))
    idx6 = jnp.stack(parts).reshape(2 * NTILES, IDXROWS_PT, 128)

    zy = jnp.zeros((GACC, D), jnp.float32)
    out = _sc_stage()(y, idx6, z, zy)

    valid = jnp.arange(G) < num_graphs
    return jnp.where(valid[:, None], out, jnp.zeros_like(out))


# speedup vs baseline: 1.8057x; 1.0813x over previous
import numpy as np
import jax
import jax.numpy as jnp
from jax import lax
from jax.experimental import pallas as pl
from jax.experimental.pallas import tpu as pltpu
from jax.experimental.pallas import tpu_sc as plsc

D = 128
P = D * (D + 1) // 2
NK = D // 2 + 1
G = 512
N = 10000
NPAD = 10240
BT = 320

NTILES = 16
ROWS_PT = NPAD // NTILES
IDXROWS_PT = ROWS_PT // 128
GH = G // 2
GACC = 384
ZROWS_PT = GACC // NTILES

_SELU_SCALE = 1.0507009873554804934193349852946
_SELU_ALPHA = 1.6732632423543772848170429916717
_RSQRT_P = 1.0 / np.sqrt(np.float32(P))
_QROOT2 = np.float32(2.0) ** 0.25

_iu, _ju = np.triu_indices(D)
_pair = np.zeros((D, D), np.int32)
_pair[_iu, _ju] = np.arange(P, dtype=np.int32)
_pair[_ju, _iu] = np.arange(P, dtype=np.int32)
_ii = np.tile(np.arange(D)[None, :], (NK, 1))
_jj = (_ii + np.arange(NK)[:, None]) % D
_PIDX = _pair[_ii, _jj]
_MASKK = np.ones((NK, D), np.float32)
_MASKK[NK - 1, D // 2:] = 0.0
_BIGIDX = np.concatenate([_PIDX, _pair], axis=0)

_TILE_OF = (np.arange(NPAD) // ROWS_PT).astype(np.int32)


def _dense_body(f_ref, idx_ref, wk_ref, a2_ref, y_ref, z_ref, zacc_ref):
    i = pl.program_id(0)
    nsteps = pl.num_programs(0)
    F = f_ref[...]
    Fs = F * _QROOT2

    acc2d = jnp.zeros((BT, D), jnp.float32)
    for k in range(1, NK):
        Fr = jnp.concatenate([Fs[:, k:], Fs[:, :k]], axis=1)
        T = Fs * Fr
        E = jnp.exp(jnp.minimum(T, 0.0))
        acc2d = acc2d + E * wk_ref[k:k + 1, :]
    Fa = jnp.abs(F)
    qf1 = jnp.sum(jnp.dot(F, a2_ref[...], preferred_element_type=jnp.float32)
                  * F, axis=1, keepdims=True)
    qf2 = jnp.sum(jnp.dot(Fa, a2_ref[...], preferred_element_type=jnp.float32)
                  * Fa, axis=1, keepdims=True)
    const = jnp.sum(wk_ref[...])
    logit = qf1 + qf2 + jnp.sum(acc2d, axis=1, keepdims=True) - const
    row = i * BT + lax.broadcasted_iota(jnp.int32, (BT, 1), 0)
    e = jnp.where(row < N, jnp.exp(logit), 0.0)
    y_ref[...] = F * e

    idx = idx_ref[...].reshape(1, BT)
    onehot_t = (lax.broadcasted_iota(jnp.int32, (G, BT), 0) == idx
                ).astype(jnp.float32)

    @pl.when(i == 0)
    def _init():
        zacc_ref[...] = jnp.zeros_like(zacc_ref)

    zacc_ref[...] += jnp.dot(onehot_t, jnp.broadcast_to(e, (BT, D)),
                             preferred_element_type=jnp.float32)

    @pl.when(i == nsteps - 1)
    def _finish():
        z_ref[...] = zacc_ref[...]


def _dense_stage(node_ft_pad, idx3, wk, a2):
    return pl.pallas_call(
        _dense_body,
        grid=(NPAD // BT,),
        in_specs=[
            pl.BlockSpec((BT, D), lambda i: (i, 0)),
            pl.BlockSpec((1, 1, BT), lambda i: (i, 0, 0)),
            pl.BlockSpec((72, D), lambda i: (0, 0)),
            pl.BlockSpec((D, D), lambda i: (0, 0)),
        ],
        out_specs=[
            pl.BlockSpec((BT, D), lambda i: (i, 0)),
            pl.BlockSpec((G, D), lambda i: (0, 0)),
        ],
        out_shape=[
            jax.ShapeDtypeStruct((NPAD, D), jnp.float32),
            jax.ShapeDtypeStruct((G, D), jnp.float32),
        ],
        scratch_shapes=[pltpu.VMEM((G, D), jnp.float32)],
    )(node_ft_pad, idx3, wk, a2)


def _sc_body(y_hbm, idx_hbm, z_hbm, zy_hbm, out_hbm,
             y_v, idx_v, accy_gather, z_gather, out_v, accy):
    c = lax.axis_index("c")
    s = lax.axis_index("s")
    glo = c * GH

    pltpu.sync_copy(zy_hbm.at[pl.ds(s * ZROWS_PT, ZROWS_PT)],
                    accy.at[pl.ds(s * ZROWS_PT, ZROWS_PT)])
    pltpu.sync_copy(idx_hbm.at[c * NTILES + s], idx_v)

    plsc.subcore_barrier()

    for j in range(IDXROWS_PT):
        pltpu.sync_copy(y_hbm.at[pl.ds(s * ROWS_PT + 128 * j, 128)], y_v)
        pltpu.sync_copy(y_v, accy.at[idx_v.at[j]], add=True)

    plsc.subcore_barrier()

    pltpu.sync_copy(accy.at[pl.ds(s * 16, 16)], accy_gather)
    pltpu.sync_copy(z_hbm.at[pl.ds(glo + s * 16, 16)], z_gather)
    for r in range(16):
        for q in range(8):
            z = jnp.maximum(z_gather[r, pl.ds(16 * q, 16)], 1e-30)
            out_v[r, pl.ds(16 * q, 16)] = accy_gather[r, pl.ds(16 * q, 16)] / z
    pltpu.sync_copy(out_v, out_hbm.at[pl.ds(glo + s * 16, 16)])


def _sc_stage():
    return pl.kernel(
        _sc_body,
        mesh=plsc.VectorSubcoreMesh(core_axis_name="c", subcore_axis_name="s"),
        out_type=jax.ShapeDtypeStruct((G, D), jnp.float32),
        scratch_types=[
            pltpu.VMEM((128, D), jnp.float32),
            pltpu.VMEM((IDXROWS_PT, 128), jnp.int32),
            pltpu.VMEM((16, D), jnp.float32),
            pltpu.VMEM((16, D), jnp.float32),
            pltpu.VMEM((16, D), jnp.float32),
            pltpu.VMEM_SHARED((GACC, D), jnp.float32),
        ],
    )


def _prep_weights(W):
    mask0 = np.copy(_MASKK)
    mask0[0] = 0.0
    wk = W[_PIDX] * jnp.asarray(mask0) * (_SELU_SCALE * _SELU_ALPHA
                                          * _RSQRT_P)
    wk = jnp.pad(wk, ((0, 72 - NK), (0, 0)))
    coefm = jnp.asarray(np.where(np.eye(D, dtype=bool), 2.0,
                                 np.sqrt(2.0)).astype(np.float32))
    a2 = W[jnp.asarray(_pair)] * coefm * np.float32(
        _SELU_SCALE * _RSQRT_P / 4.0)
    return wk, a2


def kernel(node_ft, batch_index, num_graphs, W):
    wk, a2 = _prep_weights(W)
    f_pad = jnp.pad(node_ft, ((0, NPAD - N), (0, 0)))
    idx = batch_index.astype(jnp.int32)
    idx_pad = jnp.pad(idx, (0, NPAD - N), constant_values=G)
    idx3 = jnp.where(idx_pad < G, idx_pad, 0).reshape(NPAD // BT, 1, BT)

    y, z = _dense_stage(f_pad, idx3, wk, a2)

    tile_of = jnp.asarray(_TILE_OF)
    parts = []
    for c in range(2):
        loc = idx_pad - c * GH
        ok = (loc >= 0) & (loc < GH)
        parts.append(jnp.where(ok, loc, GH + tile_of))
    idx6 = jnp.stack(parts).reshape(2 * NTILES, IDXROWS_PT, 128)

    zy = jnp.zeros((GACC, D), jnp.float32)
    out = _sc_stage()(y, idx6, z, zy)

    valid = jnp.arange(G) < num_graphs
    return jnp.where(valid[:, None], out, jnp.zeros_like(out))
